# baseline (speedup 1.0000x reference)
import functools

import jax
import jax.numpy as jnp
from jax import lax
from jax.experimental import pallas as pl
from jax.experimental.pallas import tpu as pltpu
from jax.experimental.pallas import tpu_sc as plsc

B = 1024
L = 20
S = 10
D = 128
VS_PAD = 64
MARGIN = 1.0

NIDS = B + 2 * B * L
NW = 32
CHUNK = 128
CPW = 11
NIDS_PAD = NW * CPW * CHUNK

BB = 32
NBLK = B // BB
R2 = BB * L


def _sc_gather(table, ids_pad):
    mesh = plsc.VectorSubcoreMesh(core_axis_name="c", subcore_axis_name="s")
    ipw = CPW * CHUNK

    nbuf = 6
    lag = 3

    @functools.partial(
        pl.kernel,
        mesh=mesh,
        out_type=jax.ShapeDtypeStruct((NIDS_PAD, D), jnp.float32),
        scratch_types=[
            pltpu.VMEM((ipw,), jnp.int32),
            pltpu.VMEM((nbuf, CHUNK, D), jnp.float32),
            pltpu.SemaphoreType.DMA((nbuf,)),
            pltpu.SemaphoreType.DMA((nbuf,)),
        ],
    )
    def k(table_hbm, idx_hbm, out_hbm, idx_v, rows_v, gsem, osem):
        wid = lax.axis_index("s") * 2 + lax.axis_index("c")
        pltpu.sync_copy(idx_hbm.at[pl.ds(wid * ipw, ipw)], idx_v)

        gh = [None] * CPW
        oh = [None] * CPW

        def start_gather(g):
            b = g % nbuf
            gh[g] = pltpu.async_copy(
                table_hbm.at[idx_v.at[pl.ds(g * CHUNK, CHUNK)]],
                rows_v.at[b], gsem.at[b])

        def start_copyout(g):
            b = g % nbuf
            gh[g].wait()
            oh[g] = pltpu.async_copy(
                rows_v.at[b], out_hbm.at[pl.ds(wid * ipw + g * CHUNK, CHUNK)],
                osem.at[b])

        for g in range(CPW):
            if g - nbuf >= 0:
                oh[g - nbuf].wait()
            start_gather(g)
            if g - lag >= 0:
                start_copyout(g - lag)
        for g in range(CPW - lag, CPW):
            start_copyout(g)
        for g in range(max(0, CPW - nbuf), CPW):
            if oh[g] is not None:
                oh[g].wait()

    return k(table, ids_pad)


def _softplus(x):
    return jnp.maximum(x, 0.0) + jnp.log(1.0 + jnp.exp(-jnp.abs(x)))


def _tc_body(tok_c_ref, tok_ctx_ref, tok_neg_ref, csid_ref, ncf_ref,
             sidx_ctx_ref, px_ctx_ref, sidx_neg_ref, px_neg_ref,
             wmu_t_ref, wmu_b_ref, wsig_t_ref, wsig_b_ref,
             b_mu_ref, b_sig_ref, sec_pad_ref, sec_t_ref, out_ref):
    i = pl.program_id(0)
    f32 = jnp.float32
    bi = lax.broadcasted_iota

    wmu_t = wmu_t_ref[...]
    wmu_t16 = wmu_t.astype(jnp.bfloat16)
    wsig_t = wsig_t_ref[...]
    b_mu = b_mu_ref[...]
    SP16 = S
    G640 = S * VS_PAD

    def mm(a, b):
        return jnp.dot(a, b, preferred_element_type=f32)

    sec_mu_tab = mm(sec_pad_ref[...], wmu_b_ref[...])
    tab2 = jnp.concatenate([sec_mu_tab, sec_mu_tab], axis=0)
    sec_sig_row = mm(wsig_b_ref[...], sec_t_ref[...]) + b_sig_ref[...]
    secs640 = jnp.concatenate([sec_sig_row] * S, axis=1)

    exp_m = (bi(jnp.int32, (SP16, G640), 0)
             == bi(jnp.int32, (SP16, G640), 1) // VS_PAD).astype(f32)
    sel_m = (bi(jnp.int32, (G640, SP16), 0) // VS_PAD
             == bi(jnp.int32, (G640, SP16), 1)).astype(f32)
    viota = (bi(jnp.int32, (1, G640), 1) % VS_PAD).astype(f32)
    ones16 = jnp.ones((SP16, 1), f32)
    onesD = jnp.ones((D, 1), f32)

    tok_c = tok_c_ref[...]
    csid = csid_ref[...]
    hot_c = (csid == bi(jnp.int32, (BB, VS_PAD), 1)).astype(f32)
    mu_c = mm(tok_c, wmu_t) + mm(hot_c, sec_mu_tab) + b_mu
    tsig_c = mm(tok_c, wsig_t)
    csig = mm(hot_c * sec_sig_row, jnp.ones((VS_PAD, 1), f32))
    sq = _softplus(tsig_c + csig)

    rowb = bi(jnp.int32, (R2, 1), 0) // L
    rep = (rowb == bi(jnp.int32, (R2, BB), 1)).astype(f32)
    mu_cf = mm(rep, mu_c)
    sqf = mm(rep, sq)
    ncf = mm(rep, ncf_ref[...])

    def marginal(tok_ref, sidx_ref, px_ref):
        tok = tok_ref[...]
        tmu = mm(tok.astype(jnp.bfloat16), wmu_t16)
        tsig = mm(tok, wsig_t)
        sidx = sidx_ref[...]
        px = px_ref[...]
        sid_spread = mm(sidx, exp_m)
        p_spread = mm(px, exp_m)
        eq = sid_spread == viota
        ohp640 = jnp.where(eq, p_spread, 0.0)
        hotss = jnp.where(eq, secs640, 0.0)
        ssig16 = mm(hotss, sel_m)
        ohp = (ohp640[:, 0:128] + ohp640[:, 128:256]
               + ohp640[:, 256:384] + ohp640[:, 384:512]
               + ohp640[:, 512:640])
        mu = tmu + b_mu + mm(ohp, tab2)
        ts16 = mm(tsig, jnp.ones((1, SP16), f32))
        sig = mm(px * _softplus(ts16 + ssig16), ones16) + 0.001
        return mu, sig

    mu_p, sig_p = marginal(tok_ctx_ref, sidx_ctx_ref, px_ctx_ref)
    mu_n, sig_n = marginal(tok_neg_ref, sidx_neg_ref, px_neg_ref)

    dsq = D * sqf * sqf
    dp = mu_cf - mu_p
    dn = mu_cf - mu_n
    ssp = mm(dp * dp, onesD)
    ssn = mm(dn * dn, onesD)
    kld = (D * (jnp.log(sig_p) - jnp.log(sig_n))
           + (dsq + ssp) / (2.0 * sig_p * sig_p)
           - (dsq + ssn) / (2.0 * sig_n * sig_n))
    hinge = jnp.maximum(kld + MARGIN, 0.0)
    lidx = (bi(jnp.int32, (R2, 1), 0) % L).astype(f32)
    hinge = jnp.where(lidx >= ncf, 0.0, hinge)

    @pl.when(i == 0)
    def _():
        out_ref[...] = jnp.zeros((1, 1), f32)

    out_ref[...] += jnp.sum(hinge, keepdims=True) * (1.0 / B)


def kernel(center_ids, center_section_ids, context_ids, context_section_ids,
           neg_ids, neg_section_ids, num_contexts, context_section_p,
           neg_section_p, token_emb, section_emb, W_mu, b_mu, W_sigma,
           b_sigma):
    i32 = jnp.int32
    f32 = jnp.float32

    ids_all = jnp.concatenate([
        context_ids.reshape(-1).astype(i32),
        neg_ids.reshape(-1).astype(i32),
        center_ids.astype(i32),
        jnp.zeros((NIDS_PAD - NIDS,), i32),
    ])

    rows = _sc_gather(token_emb.astype(f32), ids_all)

    csid2 = center_section_ids.astype(i32).reshape(B, 1)
    ncf2 = num_contexts.astype(f32).reshape(B, 1)
    sidx_ctx = context_section_ids.astype(f32).reshape(B * L, S)
    px_ctx = context_section_p.astype(f32).reshape(B * L, S)
    sidx_neg = neg_section_ids.astype(f32).reshape(B * L, S)
    px_neg = neg_section_p.astype(f32).reshape(B * L, S)

    wmu_t = W_mu[:D]
    wmu_b = W_mu[D:]
    wsig_t = W_sigma[:D]
    wsig_b = W_sigma[D:, 0].reshape(1, D)
    b_mu2 = b_mu.reshape(1, D)
    b_sig2 = b_sigma.reshape(1, 1)
    sec_pad = jnp.concatenate(
        [section_emb.astype(f32),
         jnp.zeros((VS_PAD - section_emb.shape[0], D), f32)], axis=0)
    sec_t = jnp.transpose(sec_pad)

    full = lambda shape: pl.BlockSpec(shape, lambda i: (0,) * len(shape))
    out = pl.pallas_call(
        _tc_body,
        grid=(NBLK,),
        in_specs=[
            pl.BlockSpec((BB, D), lambda i: (2 * B * L // BB + i, 0)),
            pl.BlockSpec((R2, D), lambda i: (i, 0)),
            pl.BlockSpec((R2, D), lambda i: (B * L // R2 + i, 0)),
            pl.BlockSpec((BB, 1), lambda i: (i, 0)),
            pl.BlockSpec((BB, 1), lambda i: (i, 0)),
            pl.BlockSpec((R2, S), lambda i: (i, 0)),
            pl.BlockSpec((R2, S), lambda i: (i, 0)),
            pl.BlockSpec((R2, S), lambda i: (i, 0)),
            pl.BlockSpec((R2, S), lambda i: (i, 0)),
            full((D, D)),
            full((D, D)),
            full((D, 1)),
            full((1, D)),
            full((1, D)),
            full((1, 1)),
            full((VS_PAD, D)),
            full((D, VS_PAD)),
        ],
        out_specs=pl.BlockSpec((1, 1), lambda i: (0, 0)),
        out_shape=jax.ShapeDtypeStruct((1, 1), f32),
    )(rows, rows, rows, csid2, ncf2, sidx_ctx, px_ctx, sidx_neg, px_neg,
      wmu_t, wmu_b, wsig_t, wsig_b, b_mu2, b_sig2, sec_pad, sec_t)

    return out[0, 0]


# speedup vs baseline: 1.0970x; 1.0970x over previous
import functools

import jax
import jax.numpy as jnp
from jax import lax
from jax.experimental import pallas as pl
from jax.experimental.pallas import tpu as pltpu
from jax.experimental.pallas import tpu_sc as plsc

B = 1024
L = 20
S = 10
D = 128
VS_PAD = 64
MARGIN = 1.0

NIDS = B + 2 * B * L
NW = 32
CHUNK = 128
CPW = 11
NIDS_PAD = NW * CPW * CHUNK

BB = 64
NBLK = B // BB
R2 = BB * L


def _sc_gather(table, ids_pad):
    mesh = plsc.VectorSubcoreMesh(core_axis_name="c", subcore_axis_name="s")
    ipw = CPW * CHUNK

    nbuf = 6
    lag = 3

    @functools.partial(
        pl.kernel,
        mesh=mesh,
        out_type=jax.ShapeDtypeStruct((NIDS_PAD, D), jnp.float32),
        scratch_types=[
            pltpu.VMEM((ipw,), jnp.int32),
            pltpu.VMEM((nbuf, CHUNK, D), jnp.float32),
            pltpu.SemaphoreType.DMA((nbuf,)),
            pltpu.SemaphoreType.DMA((nbuf,)),
        ],
    )
    def k(table_hbm, idx_hbm, out_hbm, idx_v, rows_v, gsem, osem):
        wid = lax.axis_index("s") * 2 + lax.axis_index("c")
        pltpu.sync_copy(idx_hbm.at[pl.ds(wid * ipw, ipw)], idx_v)

        gh = [None] * CPW
        oh = [None] * CPW

        def start_gather(g):
            b = g % nbuf
            gh[g] = pltpu.async_copy(
                table_hbm.at[idx_v.at[pl.ds(g * CHUNK, CHUNK)]],
                rows_v.at[b], gsem.at[b])

        def start_copyout(g):
            b = g % nbuf
            gh[g].wait()
            oh[g] = pltpu.async_copy(
                rows_v.at[b], out_hbm.at[pl.ds(wid * ipw + g * CHUNK, CHUNK)],
                osem.at[b])

        for g in range(CPW):
            if g - nbuf >= 0:
                oh[g - nbuf].wait()
            start_gather(g)
            if g - lag >= 0:
                start_copyout(g - lag)
        for g in range(CPW - lag, CPW):
            start_copyout(g)
        for g in range(max(0, CPW - nbuf), CPW):
            if oh[g] is not None:
                oh[g].wait()

    return k(table, ids_pad)


def _softplus(x):
    return jnp.maximum(x, 0.0) + jnp.log(1.0 + jnp.exp(-jnp.abs(x)))


def _tc1_body(csid_ref, ncf_ref, sidx_ctx_ref, px_ctx_ref,
              sidx_neg_ref, px_neg_ref, wmu_b_ref, wsig_b_ref,
              b_mu_ref, b_sig_ref, sec_pad_ref, sec_t_ref,
              musec_ctx_ref, musec_neg_ref, ssig_ctx_ref, ssig_neg_ref,
              cen_ref):
    f32 = jnp.float32
    bi = lax.broadcasted_iota
    G640 = S * VS_PAD

    def mm(a, b):
        return jnp.dot(a, b, preferred_element_type=f32)

    sec_mu_tab = mm(sec_pad_ref[...], wmu_b_ref[...])
    tab2 = jnp.concatenate([sec_mu_tab, sec_mu_tab], axis=0)
    sec_sig_row = mm(wsig_b_ref[...], sec_t_ref[...]) + b_sig_ref[...]
    secs640 = jnp.concatenate([sec_sig_row] * S, axis=1)
    b_mu = b_mu_ref[...]

    exp_m = (bi(jnp.int32, (S, G640), 0)
             == bi(jnp.int32, (S, G640), 1) // VS_PAD).astype(f32)
    sel_m = (bi(jnp.int32, (G640, S), 0) // VS_PAD
             == bi(jnp.int32, (G640, S), 1)).astype(f32)
    viota = (bi(jnp.int32, (1, G640), 1) % VS_PAD).astype(f32)

    def marginal(sidx_ref, px_ref, musec_ref, ssig_ref):
        sidx = sidx_ref[...]
        px = px_ref[...]
        sid_spread = mm(sidx, exp_m)
        p_spread = mm(px, exp_m)
        eq = sid_spread == viota
        ohp640 = jnp.where(eq, p_spread, 0.0)
        hotss = jnp.where(eq, secs640, 0.0)
        ssig_ref[...] = mm(hotss, sel_m)
        ohp = (ohp640[:, 0:128] + ohp640[:, 128:256]
               + ohp640[:, 256:384] + ohp640[:, 384:512]
               + ohp640[:, 512:640])
        musec_ref[...] = b_mu + mm(ohp, tab2)

    marginal(sidx_ctx_ref, px_ctx_ref, musec_ctx_ref, ssig_ctx_ref)
    marginal(sidx_neg_ref, px_neg_ref, musec_neg_ref, ssig_neg_ref)

    csid = csid_ref[...]
    hot_c = (csid == bi(jnp.int32, (BB, VS_PAD), 1)).astype(f32)
    cen_ref[:, 0:D] = mm(hot_c, sec_mu_tab)
    cen_ref[:, D:D + 1] = mm(hot_c * sec_sig_row,
                             jnp.ones((VS_PAD, 1), f32))
    cen_ref[:, D + 1:D + 2] = ncf_ref[...]


def _tc2_body(tok_c_ref, tok_ctx_ref, tok_neg_ref,
              musec_ctx_ref, musec_neg_ref, ssig_ctx_ref, ssig_neg_ref,
              px_ctx_ref, px_neg_ref, cen_ref,
              wmu_t_ref, wsig_t_ref, b_mu_ref, out_ref):
    i = pl.program_id(0)
    f32 = jnp.float32
    bi = lax.broadcasted_iota

    def mm(a, b):
        return jnp.dot(a, b, preferred_element_type=f32)

    wmu_t16 = wmu_t_ref[...].astype(jnp.bfloat16)
    wsig_t = wsig_t_ref[...]
    onesD = jnp.ones((D, 1), f32)
    onesS = jnp.ones((S, 1), f32)

    tok_c = tok_c_ref[...]
    cen = cen_ref[...]
    mu_c = (mm(tok_c.astype(jnp.bfloat16), wmu_t16)
            + cen[:, 0:D] + b_mu_ref[...])
    sq = _softplus(mm(tok_c, wsig_t) + cen[:, D:D + 1])

    rowb = bi(jnp.int32, (R2, 1), 0) // L
    rep = (rowb == bi(jnp.int32, (R2, BB), 1)).astype(f32)
    mu_cf = mm(rep, mu_c)
    sqf = mm(rep, sq)
    ncf = mm(rep, cen[:, D + 1:D + 2])

    def finish(tok_ref, musec_ref, ssig_ref, px_ref):
        tok = tok_ref[...]
        mu = mm(tok.astype(jnp.bfloat16), wmu_t16) + musec_ref[...]
        tsig = mm(tok, wsig_t)
        ts = mm(tsig, jnp.ones((1, S), f32))
        sig = mm(px_ref[...] * _softplus(ts + ssig_ref[...]), onesS) + 0.001
        return mu, sig

    mu_p, sig_p = finish(tok_ctx_ref, musec_ctx_ref, ssig_ctx_ref, px_ctx_ref)
    mu_n, sig_n = finish(tok_neg_ref, musec_neg_ref, ssig_neg_ref, px_neg_ref)

    dsq = D * sqf * sqf
    dp = mu_cf - mu_p
    dn = mu_cf - mu_n
    ssp = mm(dp * dp, onesD)
    ssn = mm(dn * dn, onesD)
    kld = (D * (jnp.log(sig_p) - jnp.log(sig_n))
           + (dsq + ssp) / (2.0 * sig_p * sig_p)
           - (dsq + ssn) / (2.0 * sig_n * sig_n))
    hinge = jnp.maximum(kld + MARGIN, 0.0)
    lidx = (bi(jnp.int32, (R2, 1), 0) % L).astype(f32)
    hinge = jnp.where(lidx >= ncf, 0.0, hinge)

    @pl.when(i == 0)
    def _():
        out_ref[...] = jnp.zeros((1, 1), f32)

    out_ref[...] += jnp.sum(hinge, keepdims=True) * (1.0 / B)


def kernel(center_ids, center_section_ids, context_ids, context_section_ids,
           neg_ids, neg_section_ids, num_contexts, context_section_p,
           neg_section_p, token_emb, section_emb, W_mu, b_mu, W_sigma,
           b_sigma):
    i32 = jnp.int32
    f32 = jnp.float32

    ids_all = jnp.concatenate([
        context_ids.reshape(-1).astype(i32),
        neg_ids.reshape(-1).astype(i32),
        center_ids.astype(i32),
        jnp.zeros((NIDS_PAD - NIDS,), i32),
    ])

    rows = _sc_gather(token_emb.astype(f32), ids_all)

    csid2 = center_section_ids.astype(i32).reshape(B, 1)
    ncf2 = num_contexts.astype(f32).reshape(B, 1)
    sidx_ctx = context_section_ids.astype(f32).reshape(B * L, S)
    px_ctx = context_section_p.astype(f32).reshape(B * L, S)
    sidx_neg = neg_section_ids.astype(f32).reshape(B * L, S)
    px_neg = neg_section_p.astype(f32).reshape(B * L, S)

    wmu_t = W_mu[:D]
    wmu_b = W_mu[D:]
    wsig_t = W_sigma[:D]
    wsig_b = W_sigma[D:, 0].reshape(1, D)
    b_mu2 = b_mu.reshape(1, D)
    b_sig2 = b_sigma.reshape(1, 1)
    sec_pad = jnp.concatenate(
        [section_emb.astype(f32),
         jnp.zeros((VS_PAD - section_emb.shape[0], D), f32)], axis=0)
    sec_t = jnp.transpose(sec_pad)

    full = lambda shape: pl.BlockSpec(shape, lambda i: (0,) * len(shape))
    bspec = lambda shape: pl.BlockSpec(shape, lambda i: (i, 0))

    musec_ctx, musec_neg, ssig_ctx, ssig_neg, cen = pl.pallas_call(
        _tc1_body,
        grid=(NBLK,),
        in_specs=[
            bspec((BB, 1)), bspec((BB, 1)),
            bspec((R2, S)), bspec((R2, S)), bspec((R2, S)), bspec((R2, S)),
            full((D, D)), full((1, D)), full((1, D)), full((1, 1)),
            full((VS_PAD, D)), full((D, VS_PAD)),
        ],
        out_specs=[bspec((R2, D)), bspec((R2, D)),
                   bspec((R2, S)), bspec((R2, S)), bspec((BB, D + 2))],
        out_shape=[
            jax.ShapeDtypeStruct((B * L, D), f32),
            jax.ShapeDtypeStruct((B * L, D), f32),
            jax.ShapeDtypeStruct((B * L, S), f32),
            jax.ShapeDtypeStruct((B * L, S), f32),
            jax.ShapeDtypeStruct((B, D + 2), f32),
        ],
    )(csid2, ncf2, sidx_ctx, px_ctx, sidx_neg, px_neg,
      wmu_b, wsig_b, b_mu2, b_sig2, sec_pad, sec_t)

    out = pl.pallas_call(
        _tc2_body,
        grid=(NBLK,),
        in_specs=[
            pl.BlockSpec((BB, D), lambda i: (2 * B * L // BB + i, 0)),
            pl.BlockSpec((R2, D), lambda i: (i, 0)),
            pl.BlockSpec((R2, D), lambda i: (B * L // R2 + i, 0)),
            bspec((R2, D)), bspec((R2, D)),
            bspec((R2, S)), bspec((R2, S)),
            bspec((R2, S)), bspec((R2, S)),
            bspec((BB, D + 2)),
            full((D, D)), full((D, 1)), full((1, D)),
        ],
        out_specs=pl.BlockSpec((1, 1), lambda i: (0, 0)),
        out_shape=jax.ShapeDtypeStruct((1, 1), f32),
    )(rows, rows, rows, musec_ctx, musec_neg, ssig_ctx, ssig_neg,
      px_ctx, px_neg, cen, wmu_t, wsig_t, b_mu2)

    return out[0, 0]
